# Initial kernel scaffold; baseline (speedup 1.0000x reference)
import jax
import jax.numpy as jnp
from jax.experimental import pallas as pl


def kernel(x, edge_index, W1, b1, W2, b2, Wl, bl):
    raise NotImplementedError("write your pallas kernel here")


# speedup vs baseline: 216.4965x; 216.4965x over previous
import functools

import jax
import jax.numpy as jnp
from jax import lax
from jax.experimental import pallas as pl
from jax.experimental.pallas import tpu as pltpu
from jax.experimental.pallas import tpu_sc as plsc

N = 100000
E = 6400000
NP = 100352
ROWS = E // 128
NC, NS, LANES = 2, 16, 16
NW = NC * NS
RPW = ROWS // NW
REM = ROWS - RPW * NW
CH = 32
NCH = (RPW + 1 + CH - 1) // CH
SEG = NP // NS

def _mesh():
    return plsc.VectorSubcoreMesh(core_axis_name="c", subcore_axis_name="s",
                                  num_cores=NC, num_subcores=NS)


def _edge_range(wid):
    n = RPW + jnp.where(wid < REM, 1, 0).astype(jnp.int32)
    base = wid * RPW + jnp.minimum(wid, REM)
    return base, n


def _zero_fill(ref, nwords):
    zv = jnp.zeros((LANES,), jnp.float32)

    @pl.loop(0, nwords // LANES)
    def _(i):
        ref[pl.ds(i * LANES, LANES)] = zv


def _hist_body(dst_h, out_h, zbuf, bigbuf, idxrow, ones, hist_sh):
    c = lax.axis_index("c")
    s = lax.axis_index("s")
    wid = s * NC + c
    _zero_fill(zbuf, SEG)
    ov = jnp.ones((LANES,), jnp.float32)

    @pl.loop(0, 128 // LANES)
    def _(i):
        ones[pl.ds(i * LANES, LANES)] = ov

    pltpu.sync_copy(zbuf, hist_sh.at[pl.ds(s * SEG, SEG)])
    plsc.subcore_barrier()
    base, n = _edge_range(wid)

    @pl.loop(0, NCH)
    def _(ch):
        start = base + ch * CH
        nrows = jnp.minimum(CH, n - ch * CH)
        rst = jnp.minimum(start, ROWS - CH)
        off = start - rst
        pltpu.sync_copy(dst_h.at[pl.ds(rst * 128, CH * 128)], bigbuf)

        @pl.loop(0, nrows)
        def _(j):
            b = (off + j) * 128
            for k in range(8):
                idxrow[pl.ds(k * LANES, LANES)] = bigbuf[pl.ds(b + k * LANES, LANES)]
            pltpu.sync_copy(ones, hist_sh.at[idxrow], add=True)

    plsc.subcore_barrier()
    pltpu.sync_copy(hist_sh.at[pl.ds(s * SEG, SEG)],
                    out_h.at[c, pl.ds(s * SEG, SEG)])


def _hist(dst):
    return pl.kernel(
        _hist_body,
        out_type=jax.ShapeDtypeStruct((NC, NP), jnp.float32),
        mesh=_mesh(),
        compiler_params=pltpu.CompilerParams(needs_layout_passes=False),
        scratch_types=[
            pltpu.VMEM((SEG,), jnp.float32),
            pltpu.VMEM((CH * 128,), jnp.int32),
            pltpu.VMEM((128,), jnp.int32),
            pltpu.VMEM((128,), jnp.float32),
            pltpu.VMEM_SHARED((NP,), jnp.float32),
        ],
    )(dst)


def _agg_body(src_h, dst_h, v_h, out_h, vtab, zbuf, bigs, bigd, drow, valrow,
              s1_sh):
    c = lax.axis_index("c")
    s = lax.axis_index("s")
    wid = s * NC + c
    pltpu.sync_copy(v_h, vtab)
    _zero_fill(zbuf, SEG)
    pltpu.sync_copy(zbuf, s1_sh.at[pl.ds(s * SEG, SEG)])
    plsc.subcore_barrier()
    base, n = _edge_range(wid)

    @pl.loop(0, NCH)
    def _(ch):
        start = base + ch * CH
        nrows = jnp.minimum(CH, n - ch * CH)
        rst = jnp.minimum(start, ROWS - CH)
        off = start - rst
        pltpu.sync_copy(src_h.at[pl.ds(rst * 128, CH * 128)], bigs)
        pltpu.sync_copy(dst_h.at[pl.ds(rst * 128, CH * 128)], bigd)

        @pl.loop(0, nrows)
        def _(j):
            b = (off + j) * 128
            for k in range(8):
                sl = pl.ds(k * LANES, LANES)
                iv = bigs[pl.ds(b + k * LANES, LANES)]
                valrow[sl] = plsc.load_gather(vtab, [iv])
                drow[sl] = bigd[pl.ds(b + k * LANES, LANES)]
            pltpu.sync_copy(valrow, s1_sh.at[drow], add=True)

    plsc.subcore_barrier()
    pltpu.sync_copy(s1_sh.at[pl.ds(s * SEG, SEG)],
                    out_h.at[c, pl.ds(s * SEG, SEG)])


def _agg(src, dst, v):
    return pl.kernel(
        _agg_body,
        out_type=jax.ShapeDtypeStruct((NC, NP), jnp.float32),
        mesh=_mesh(),
        compiler_params=pltpu.CompilerParams(needs_layout_passes=False),
        scratch_types=[
            pltpu.VMEM((NP,), jnp.float32),
            pltpu.VMEM((SEG,), jnp.float32),
            pltpu.VMEM((CH * 128,), jnp.int32),
            pltpu.VMEM((CH * 128,), jnp.int32),
            pltpu.VMEM((128,), jnp.int32),
            pltpu.VMEM((128,), jnp.float32),
            pltpu.VMEM_SHARED((NP,), jnp.float32),
        ],
    )(src, dst, v)


def _dot_body(src_h, dst_h, w_h, out_h, wtab, bigs, bigd, acc):
    c = lax.axis_index("c")
    s = lax.axis_index("s")
    wid = s * NC + c
    pltpu.sync_copy(w_h, wtab)
    acc[...] = jnp.zeros((LANES,), jnp.float32)
    base, n = _edge_range(wid)
    himask = jnp.int32(-65536)

    @pl.loop(0, NCH)
    def _(ch):
        start = base + ch * CH
        nrows = jnp.minimum(CH, n - ch * CH)
        rst = jnp.minimum(start, ROWS - CH)
        off = start - rst
        pltpu.sync_copy(src_h.at[pl.ds(rst * 128, CH * 128)], bigs)
        pltpu.sync_copy(dst_h.at[pl.ds(rst * 128, CH * 128)], bigd)

        @pl.loop(0, nrows)
        def _(j):
            b = (off + j) * 128
            for k in range(8):
                si = bigs[pl.ds(b + k * LANES, LANES)]
                di = bigd[pl.ds(b + k * LANES, LANES)]
                ws = plsc.load_gather(wtab, [si])
                wd = plsc.load_gather(wtab, [di])
                u = plsc.bitcast(jnp.bitwise_and(ws, himask), jnp.float32)
                a = plsc.bitcast(jnp.left_shift(wd, 16), jnp.float32)
                acc[...] = acc[...] + u * a

    pltpu.sync_copy(acc, out_h.at[wid])


def _dot(src, dst, wpk):
    return pl.kernel(
        _dot_body,
        out_type=jax.ShapeDtypeStruct((NW, LANES), jnp.float32),
        mesh=_mesh(),
        compiler_params=pltpu.CompilerParams(needs_layout_passes=False),
        scratch_types=[
            pltpu.VMEM((NP,), jnp.int32),
            pltpu.VMEM((CH * 128,), jnp.int32),
            pltpu.VMEM((CH * 128,), jnp.int32),
            pltpu.VMEM((LANES,), jnp.float32),
        ],
    )(src, dst, wpk)


def _node1_body(hist_ref, x_ref, dinv_ref, v_ref, xod_ref):
    h = hist_ref[...]
    deg = h[0] + h[1] + 1.0
    dinv = lax.rsqrt(deg)
    x = x_ref[...]
    dinv_ref[...] = dinv
    v_ref[...] = x * dinv
    xod_ref[...] = x * dinv * dinv


def _node1(histp, x_p):
    return pl.pallas_call(
        _node1_body,
        out_shape=[jax.ShapeDtypeStruct((784, 128), jnp.float32)] * 3,
    )(histp, x_p)


def _node2_body(s1p_ref, dinv_ref, xod_ref, wl_ref, w1_ref, b1_ref,
                wpk_ref, sself_ref, swl_ref):
    s1 = s1p_ref[0] + s1p_ref[1]
    dinv = dinv_ref[...]
    w1 = w1_ref[0, 0]
    b1 = b1_ref[0, 0]
    h1 = w1 * (dinv * s1 + xod_ref[...]) + b1
    h1r = jnp.maximum(h1, 0.0)
    wl = wl_ref[...]
    u = h1r * dinv
    a = wl * dinv
    bu = lax.bitcast_convert_type(u, jnp.int32) + 0x8000
    ba = lax.bitcast_convert_type(a, jnp.int32) + 0x8000
    wpk_ref[...] = jnp.bitwise_and(bu, jnp.int32(-65536)) | lax.shift_right_logical(ba, 16)
    sself_ref[0, 0] = jnp.sum(wl * h1r * dinv * dinv)
    swl_ref[0, 0] = jnp.sum(wl)


def _node2(s1p, dinv, xod, wl_p, W1, b1):
    return pl.pallas_call(
        _node2_body,
        in_specs=[pl.BlockSpec(memory_space=pltpu.VMEM)] * 4
        + [pl.BlockSpec(memory_space=pltpu.SMEM)] * 2,
        out_specs=[pl.BlockSpec(memory_space=pltpu.VMEM),
                   pl.BlockSpec(memory_space=pltpu.SMEM),
                   pl.BlockSpec(memory_space=pltpu.SMEM)],
        out_shape=[jax.ShapeDtypeStruct((784, 128), jnp.int32),
                   jax.ShapeDtypeStruct((1, 1), jnp.float32),
                   jax.ShapeDtypeStruct((1, 1), jnp.float32)],
    )(s1p, dinv, xod, wl_p, W1, b1)


def _final_body(part_ref, sself_ref, swl_ref, w2_ref, b2_ref, bl_ref, out_ref):
    es = jnp.sum(part_ref[...])
    out_ref[0, 0] = (w2_ref[0, 0] * (es + sself_ref[0, 0])
                     + b2_ref[0, 0] * swl_ref[0, 0] + bl_ref[0, 0])


def _final(part, sself, swl, W2, b2, bl):
    return pl.pallas_call(
        _final_body,
        in_specs=[pl.BlockSpec(memory_space=pltpu.VMEM)]
        + [pl.BlockSpec(memory_space=pltpu.SMEM)] * 5,
        out_specs=pl.BlockSpec(memory_space=pltpu.SMEM),
        out_shape=jax.ShapeDtypeStruct((1, 1), jnp.float32),
    )(part, sself, swl, W2, b2, bl)


def kernel(x, edge_index, W1, b1, W2, b2, Wl, bl):
    src = edge_index[0]
    dst = edge_index[1]
    xf = jnp.pad(x.reshape(N), (0, NP - N)).reshape(784, 128)
    wl_p = jnp.pad(Wl.reshape(N), (0, NP - N)).reshape(784, 128)

    histp = _hist(dst)
    dinv, v, xod = _node1(histp.reshape(2, 784, 128), xf)
    s1p = _agg(src, dst, v.reshape(NP))
    wpk, sself, swl = _node2(s1p.reshape(2, 784, 128), dinv, xod, wl_p,
                             W1, b1.reshape(1, 1))
    part = _dot(src, dst, wpk.reshape(NP))
    out = _final(part.reshape(4, 128), sself, swl,
                 W2, b2.reshape(1, 1), bl.reshape(1, 1))
    return out


# speedup vs baseline: 561.6119x; 2.5941x over previous
import jax
import jax.numpy as jnp
from jax import lax
from jax.experimental import pallas as pl
from jax.experimental.pallas import tpu as pltpu
from jax.experimental.pallas import tpu_sc as plsc

N = 100000
E = 6400000
NP = 100352
ROWS = E // 128
NC, NS, LANES = 2, 16, 16
NW = NC * NS
CH = 32
GR = 8
NGROUPS = ROWS // GR
GPW = NGROUPS // NW
GREM = NGROUPS - GPW * NW
NCH = ((GPW + 1) * GR + CH - 1) // CH
SEG = NP // NS


def _mesh():
    return plsc.VectorSubcoreMesh(core_axis_name="c", subcore_axis_name="s",
                                  num_cores=NC, num_subcores=NS)


def _sc_params():
    return pltpu.CompilerParams(needs_layout_passes=False)


def _edge_range(wid):
    g = GPW + jnp.where(wid < GREM, 1, 0).astype(jnp.int32)
    base_g = wid * GPW + jnp.minimum(wid, GREM)
    return base_g * GR, g * GR


def _zero_fill(ref, nwords):
    zv = jnp.zeros((LANES,), jnp.float32)

    @pl.loop(0, nwords // LANES)
    def _(i):
        ref[pl.ds(i * LANES, LANES)] = zv


def _rst_of(base, ch):
    return jnp.minimum(base + ch * CH, ROWS - CH)


def _double_buffered_chunks(base, total, start_in, wait_in, process):
    start_in(0, 0)

    @pl.loop(0, NCH // 2)
    def _(h):
        ch0 = 2 * h
        wait_in(ch0, 0)
        start_in(ch0 + 1, 1)
        process(ch0, 0)
        wait_in(ch0 + 1, 1)
        start_in(ch0 + 2, 0)
        process(ch0 + 1, 1)

    wait_in(NCH - 1, 0)
    process(NCH - 1, 0)


def _hist_body(dst_h, out_h, zbuf, biga, bigb, ones, hist_sh, sa, sb, ssc):
    c = lax.axis_index("c")
    s = lax.axis_index("s")
    wid = s * NC + c
    _zero_fill(zbuf, SEG)
    ov = jnp.ones((LANES,), jnp.float32)

    @pl.loop(0, 128 // LANES)
    def _(i):
        ones[pl.ds(i * LANES, LANES)] = ov

    pltpu.sync_copy(zbuf, hist_sh.at[pl.ds(s * SEG, SEG)])
    plsc.subcore_barrier()
    base, total = _edge_range(wid)
    bufs = (biga, bigb)
    sems = (sa, sb)

    def start_in(ch, b):
        ch = jnp.minimum(ch, NCH - 1)
        pltpu.async_copy(dst_h.at[pl.ds(_rst_of(base, ch), CH)], bufs[b],
                         sems[b])

    def wait_in(ch, b):
        pltpu.make_async_copy(dst_h.at[pl.ds(_rst_of(base, ch), CH)], bufs[b],
                              sems[b]).wait()

    def process(ch, b):
        buf = bufs[b]
        start = base + ch * CH
        off = start - _rst_of(base, ch)
        nrows = jnp.minimum(CH, total - ch * CH)
        drain = pltpu.make_async_copy(ones, hist_sh.at[buf.at[off]], ssc)

        @pl.loop(0, nrows // GR)
        def _(g):
            for i in range(GR):
                j = off + g * GR + i
                pltpu.async_copy(ones, hist_sh.at[buf.at[j]], ssc, add=True)
            for _i in range(GR):
                drain.wait()

    _double_buffered_chunks(base, total, start_in, wait_in, process)
    plsc.subcore_barrier()
    pltpu.sync_copy(hist_sh.at[pl.ds(s * SEG, SEG)],
                    out_h.at[c, pl.ds(s * SEG, SEG)])


def _hist(dst):
    return pl.kernel(
        _hist_body,
        out_type=jax.ShapeDtypeStruct((NC, NP), jnp.float32),
        mesh=_mesh(),
        compiler_params=_sc_params(),
        scratch_types=[
            pltpu.VMEM((SEG,), jnp.float32),
            pltpu.VMEM((CH, 128), jnp.int32),
            pltpu.VMEM((CH, 128), jnp.int32),
            pltpu.VMEM((128,), jnp.float32),
            pltpu.VMEM_SHARED((NP,), jnp.float32),
            pltpu.SemaphoreType.DMA,
            pltpu.SemaphoreType.DMA,
            pltpu.SemaphoreType.DMA,
        ],
    )(dst)


def _agg_body(src_h, dst_h, v_h, out_h, vtab, zbuf, bsa, bsb, bda, bdb,
              valbufs, s1_sh, ssa, ssb, sda, sdb, ssc):
    c = lax.axis_index("c")
    s = lax.axis_index("s")
    wid = s * NC + c
    pltpu.sync_copy(v_h, vtab)
    _zero_fill(zbuf, SEG)
    pltpu.sync_copy(zbuf, s1_sh.at[pl.ds(s * SEG, SEG)])
    plsc.subcore_barrier()
    base, total = _edge_range(wid)
    sbufs, dbufs = (bsa, bsb), (bda, bdb)
    ssems, dsems = (ssa, ssb), (sda, sdb)

    def start_in(ch, b):
        ch = jnp.minimum(ch, NCH - 1)
        rst = _rst_of(base, ch)
        pltpu.async_copy(src_h.at[pl.ds(rst, CH)], sbufs[b], ssems[b])
        pltpu.async_copy(dst_h.at[pl.ds(rst, CH)], dbufs[b], dsems[b])

    def wait_in(ch, b):
        rst = _rst_of(base, ch)
        pltpu.make_async_copy(src_h.at[pl.ds(rst, CH)], sbufs[b],
                              ssems[b]).wait()
        pltpu.make_async_copy(dst_h.at[pl.ds(rst, CH)], dbufs[b],
                              dsems[b]).wait()

    def process(ch, b):
        sb_, db_ = sbufs[b], dbufs[b]
        start = base + ch * CH
        off = start - _rst_of(base, ch)
        nrows = jnp.minimum(CH, total - ch * CH)
        drain = pltpu.make_async_copy(valbufs.at[0], s1_sh.at[db_.at[off]],
                                      ssc)

        @pl.loop(0, nrows // GR)
        def _(g):
            for i in range(GR):
                j = off + g * GR + i
                for k in range(8):
                    iv = sb_[j, pl.ds(k * LANES, LANES)]
                    valbufs[i, pl.ds(k * LANES, LANES)] = (
                        plsc.load_gather(vtab, [iv]))
                pltpu.async_copy(valbufs.at[i], s1_sh.at[db_.at[j]], ssc,
                                 add=True)
            for _i in range(GR):
                drain.wait()

    _double_buffered_chunks(base, total, start_in, wait_in, process)
    plsc.subcore_barrier()
    pltpu.sync_copy(s1_sh.at[pl.ds(s * SEG, SEG)],
                    out_h.at[c, pl.ds(s * SEG, SEG)])


def _agg(src, dst, v):
    return pl.kernel(
        _agg_body,
        out_type=jax.ShapeDtypeStruct((NC, NP), jnp.float32),
        mesh=_mesh(),
        compiler_params=_sc_params(),
        scratch_types=[
            pltpu.VMEM((NP,), jnp.float32),
            pltpu.VMEM((SEG,), jnp.float32),
            pltpu.VMEM((CH, 128), jnp.int32),
            pltpu.VMEM((CH, 128), jnp.int32),
            pltpu.VMEM((CH, 128), jnp.int32),
            pltpu.VMEM((CH, 128), jnp.int32),
            pltpu.VMEM((GR, 128), jnp.float32),
            pltpu.VMEM_SHARED((NP,), jnp.float32),
            pltpu.SemaphoreType.DMA,
            pltpu.SemaphoreType.DMA,
            pltpu.SemaphoreType.DMA,
            pltpu.SemaphoreType.DMA,
            pltpu.SemaphoreType.DMA,
        ],
    )(src, dst, v)


def _dot_body(src_h, dst_h, w_h, out_h, wtab, bsa, bsb, bda, bdb, acc,
              ssa, ssb, sda, sdb):
    c = lax.axis_index("c")
    s = lax.axis_index("s")
    wid = s * NC + c
    pltpu.sync_copy(w_h, wtab)
    acc[...] = jnp.zeros((LANES,), jnp.float32)
    base, total = _edge_range(wid)
    sbufs, dbufs = (bsa, bsb), (bda, bdb)
    ssems, dsems = (ssa, ssb), (sda, sdb)

    def start_in(ch, b):
        ch = jnp.minimum(ch, NCH - 1)
        rst = _rst_of(base, ch)
        pltpu.async_copy(src_h.at[pl.ds(rst, CH)], sbufs[b], ssems[b])
        pltpu.async_copy(dst_h.at[pl.ds(rst, CH)], dbufs[b], dsems[b])

    def wait_in(ch, b):
        rst = _rst_of(base, ch)
        pltpu.make_async_copy(src_h.at[pl.ds(rst, CH)], sbufs[b],
                              ssems[b]).wait()
        pltpu.make_async_copy(dst_h.at[pl.ds(rst, CH)], dbufs[b],
                              dsems[b]).wait()

    def process(ch, b):
        sb_, db_ = sbufs[b], dbufs[b]
        start = base + ch * CH
        off = start - _rst_of(base, ch)
        nrows = jnp.minimum(CH, total - ch * CH)

        @pl.loop(0, nrows)
        def _(j):
            r = off + j
            prods = []
            for k in range(8):
                si = sb_[r, pl.ds(k * LANES, LANES)]
                di = db_[r, pl.ds(k * LANES, LANES)]
                ws = plsc.load_gather(wtab, [si])
                wd = plsc.load_gather(wtab, [di])
                uq = lax.shift_right_arithmetic(ws, 16)
                aq = lax.shift_right_arithmetic(lax.shift_left(wd, 16), 16)
                prods.append(uq.astype(jnp.float32) * aq.astype(jnp.float32))
            while len(prods) > 1:
                prods = [p0 + p1 for p0, p1 in zip(prods[::2], prods[1::2])]
            acc[...] = acc[...] + prods[0]

    _double_buffered_chunks(base, total, start_in, wait_in, process)
    pltpu.sync_copy(acc, out_h.at[wid])


def _dot(src, dst, wpk):
    return pl.kernel(
        _dot_body,
        out_type=jax.ShapeDtypeStruct((NW, LANES), jnp.float32),
        mesh=_mesh(),
        compiler_params=_sc_params(),
        scratch_types=[
            pltpu.VMEM((NP,), jnp.int32),
            pltpu.VMEM((CH, 128), jnp.int32),
            pltpu.VMEM((CH, 128), jnp.int32),
            pltpu.VMEM((CH, 128), jnp.int32),
            pltpu.VMEM((CH, 128), jnp.int32),
            pltpu.VMEM((LANES,), jnp.float32),
            pltpu.SemaphoreType.DMA,
            pltpu.SemaphoreType.DMA,
            pltpu.SemaphoreType.DMA,
            pltpu.SemaphoreType.DMA,
        ],
    )(src, dst, wpk)


def _node1_body(hist_ref, x_ref, dinv_ref, v_ref, xod_ref):
    h = hist_ref[...]
    deg = h[0] + h[1] + 1.0
    dinv = lax.rsqrt(deg)
    x = x_ref[...]
    dinv_ref[...] = dinv
    v_ref[...] = x * dinv
    xod_ref[...] = x * dinv * dinv


def _node1(histp, x_p):
    return pl.pallas_call(
        _node1_body,
        out_shape=[jax.ShapeDtypeStruct((784, 128), jnp.float32)] * 3,
    )(histp, x_p)


def _node2_body(s1p_ref, dinv_ref, xod_ref, wl_ref, w1_ref, b1_ref,
                wpk_ref, sself_ref, swl_ref, ss_ref):
    s1 = s1p_ref[0] + s1p_ref[1]
    dinv = dinv_ref[...]
    w1 = w1_ref[0, 0]
    b1 = b1_ref[0, 0]
    h1 = w1 * (dinv * s1 + xod_ref[...]) + b1
    h1r = jnp.maximum(h1, 0.0)
    wl = wl_ref[...]
    u = h1r * dinv
    a = wl * dinv
    su = jnp.maximum(jnp.max(jnp.abs(u)), 1e-30)
    sa = jnp.maximum(jnp.max(jnp.abs(a)), 1e-30)
    qu = jnp.round(u * (32767.0 / su)).astype(jnp.int32)
    qa = jnp.round(a * (32767.0 / sa)).astype(jnp.int32)
    wpk_ref[...] = lax.shift_left(qu, 16) | jnp.bitwise_and(qa, jnp.int32(0xFFFF))
    sself_ref[0, 0] = jnp.sum(wl * h1r * dinv * dinv)
    swl_ref[0, 0] = jnp.sum(wl)
    ss_ref[0, 0] = su * sa / (32767.0 * 32767.0)


def _node2(s1p, dinv, xod, wl_p, W1, b1):
    return pl.pallas_call(
        _node2_body,
        in_specs=[pl.BlockSpec(memory_space=pltpu.VMEM)] * 4
        + [pl.BlockSpec(memory_space=pltpu.SMEM)] * 2,
        out_specs=[pl.BlockSpec(memory_space=pltpu.VMEM),
                   pl.BlockSpec(memory_space=pltpu.SMEM),
                   pl.BlockSpec(memory_space=pltpu.SMEM),
                   pl.BlockSpec(memory_space=pltpu.SMEM)],
        out_shape=[jax.ShapeDtypeStruct((784, 128), jnp.int32),
                   jax.ShapeDtypeStruct((1, 1), jnp.float32),
                   jax.ShapeDtypeStruct((1, 1), jnp.float32),
                   jax.ShapeDtypeStruct((1, 1), jnp.float32)],
    )(s1p, dinv, xod, wl_p, W1, b1)


def _final_body(part_ref, sself_ref, swl_ref, ss_ref, w2_ref, b2_ref, bl_ref,
                out_ref):
    es = jnp.sum(part_ref[...]) * ss_ref[0, 0]
    out_ref[0, 0] = (w2_ref[0, 0] * (es + sself_ref[0, 0])
                     + b2_ref[0, 0] * swl_ref[0, 0] + bl_ref[0, 0])


def _final(part, sself, swl, ss, W2, b2, bl):
    return pl.pallas_call(
        _final_body,
        in_specs=[pl.BlockSpec(memory_space=pltpu.VMEM)]
        + [pl.BlockSpec(memory_space=pltpu.SMEM)] * 6,
        out_specs=pl.BlockSpec(memory_space=pltpu.SMEM),
        out_shape=jax.ShapeDtypeStruct((1, 1), jnp.float32),
    )(part, sself, swl, ss, W2, b2, bl)


def kernel(x, edge_index, W1, b1, W2, b2, Wl, bl):
    src = edge_index[0].reshape(ROWS, 128)
    dst = edge_index[1].reshape(ROWS, 128)
    xf = jnp.pad(x.reshape(N), (0, NP - N)).reshape(784, 128)
    wl_p = jnp.pad(Wl.reshape(N), (0, NP - N)).reshape(784, 128)

    histp = _hist(dst)
    dinv, v, xod = _node1(histp.reshape(2, 784, 128), xf)
    s1p = _agg(src, dst, v.reshape(NP))
    wpk, sself, swl, ss = _node2(s1p.reshape(2, 784, 128), dinv, xod, wl_p,
                                 W1, b1.reshape(1, 1))
    part = _dot(src, dst, wpk.reshape(NP))
    out = _final(part.reshape(4, 128), sself, swl, ss,
                 W2, b2.reshape(1, 1), bl.reshape(1, 1))
    return out
